# initial kernel scaffold (unmeasured)
import jax
import jax.numpy as jnp
from jax import lax
from jax.experimental import pallas as pl
from jax.experimental.pallas import tpu as pltpu

B, H, D, BS = 16, 16, 64, 16
NSLOTS = 128
NPAGES_LOCAL = 128
NKEYS = NPAGES_LOCAL * BS


def kernel(Q, K, V, bt, lens):
    lens2 = lens.reshape(B, 1)

    def body(q_ref, k_ref, v_ref, bt_ref, lens_ref, out_ref,
             ml_send, ml_recv, o_send, o_recv,
             send_sem_ml, recv_sem_ml, send_sem_o, recv_sem_o):
        my_x = lax.axis_index("x")
        my_y = lax.axis_index("y")
        nbr = (1 - my_x, my_y)

        barrier = pltpu.get_barrier_semaphore()
        pl.semaphore_signal(barrier, inc=1, device_id=nbr,
                            device_id_type=pl.DeviceIdType.MESH)
        pl.semaphore_wait(barrier, 1)

        q = q_ref[:, 0, :, :].astype(jnp.bfloat16)
        k = k_ref[...].reshape(NKEYS, H, D).astype(jnp.bfloat16)
        v = v_ref[...].reshape(NKEYS, H, D).astype(jnp.bfloat16)

        s = lax.dot_general(q, k, (((2,), (2,)), ((1,), (1,))),
                            preferred_element_type=jnp.float32)
        s = s * (D ** -0.5)

        slot = lax.broadcasted_iota(jnp.int32, (B, NSLOTS, NPAGES_LOCAL), 1)
        page = lax.broadcasted_iota(jnp.int32, (B, NSLOTS, NPAGES_LOCAL), 2)
        btl = bt_ref[...] - my_x * NPAGES_LOCAL
        hit = (btl[:, :, None] == page) & (slot < lens_ref[...][:, :, None])
        counts = jnp.sum(hit.astype(jnp.float32), axis=1)

        rk = lax.broadcasted_iota(jnp.int32, (NPAGES_LOCAL, NKEYS), 1) // BS
        rp = lax.broadcasted_iota(jnp.int32, (NPAGES_LOCAL, NKEYS), 0)
        repmat = (rk == rp).astype(jnp.bfloat16)
        ck = lax.dot_general(counts.astype(jnp.bfloat16), repmat,
                             (((1,), (0,)), ((), ())),
                             preferred_element_type=jnp.float32)

        m_loc = jnp.max(s, axis=2)
        ew = jnp.exp(s - m_loc[:, :, None]) * ck[None, :, :]
        l_loc = jnp.sum(ew, axis=2)
        o_loc = lax.dot_general(ew.astype(jnp.bfloat16), v,
                                (((2,), (0,)), ((0,), (1,))),
                                preferred_element_type=jnp.float32)

        ml_send[0, :, :] = m_loc
        ml_send[1, :, :] = l_loc
        o_send[...] = o_loc

        rdma_ml = pltpu.make_async_remote_copy(
            src_ref=ml_send, dst_ref=ml_recv,
            send_sem=send_sem_ml, recv_sem=recv_sem_ml,
            device_id=nbr, device_id_type=pl.DeviceIdType.MESH)
        rdma_o = pltpu.make_async_remote_copy(
            src_ref=o_send, dst_ref=o_recv,
            send_sem=send_sem_o, recv_sem=recv_sem_o,
            device_id=nbr, device_id_type=pl.DeviceIdType.MESH)
        rdma_ml.start()
        rdma_o.start()
        rdma_ml.wait()
        rdma_o.wait()

        m_rem = ml_recv[0, :, :]
        l_rem = ml_recv[1, :, :]
        m = jnp.maximum(m_loc, m_rem)
        a_loc = jnp.exp(m_loc - m)
        a_rem = jnp.exp(m_rem - m)
        l = l_loc * a_loc + l_rem * a_rem
        o = (o_loc * a_loc[:, :, None] + o_recv[...] * a_rem[:, :, None])
        o = o / l[:, :, None]
        out_ref[...] = jnp.swapaxes(o, 0, 1).reshape(B, 1, H, D)

    return pl.pallas_call(
        body,
        out_shape=jax.ShapeDtypeStruct((B, 1, H, D), jnp.float32),
        in_specs=[pl.BlockSpec(memory_space=pltpu.VMEM)] * 5,
        out_specs=pl.BlockSpec(memory_space=pltpu.VMEM),
        scratch_shapes=[
            pltpu.VMEM((2, H, B), jnp.float32),
            pltpu.VMEM((2, H, B), jnp.float32),
            pltpu.VMEM((H, B, D), jnp.float32),
            pltpu.VMEM((H, B, D), jnp.float32),
            pltpu.SemaphoreType.DMA,
            pltpu.SemaphoreType.DMA,
            pltpu.SemaphoreType.DMA,
            pltpu.SemaphoreType.DMA,
        ],
        compiler_params=pltpu.CompilerParams(collective_id=0),
    )(Q, K, V, bt, lens2)


# baseline (device time: 65259 ns/iter reference)
import jax
import jax.numpy as jnp
from jax import lax
from jax.experimental import pallas as pl
from jax.experimental.pallas import tpu as pltpu

B, H, D, BS = 16, 16, 64, 16
NSLOTS = 128
NPAGES_LOCAL = 128
NCHUNK = 8
CP = NPAGES_LOCAL // NCHUNK
CK = CP * BS


def kernel(Q, K, V, bt, lens):
    lens2 = lens.reshape(B, 1)

    def body(q_ref, k_ref, v_ref, bt_ref, lens_ref, out_ref,
             m_acc, l_acc, o_acc, ml_send, ml_recv, o_recv,
             send_sem_ml, recv_sem_ml, send_sem_o, recv_sem_o):
        i = pl.program_id(0)
        my_x = lax.axis_index("x")
        my_y = lax.axis_index("y")
        nbr = (1 - my_x, my_y)
        barrier = pltpu.get_barrier_semaphore()

        @pl.when(i == 0)
        def _():
            pl.semaphore_signal(barrier, inc=1, device_id=nbr,
                                device_id_type=pl.DeviceIdType.MESH)
            m_acc[...] = jnp.full((H, B), -1e30, jnp.float32)
            l_acc[...] = jnp.zeros((H, B), jnp.float32)
            o_acc[...] = jnp.zeros((H, B, D), jnp.float32)

        q = q_ref[:, 0, :, :].astype(jnp.bfloat16)
        k = k_ref[...].reshape(CK, H, D).astype(jnp.bfloat16)
        v = v_ref[...].reshape(CK, H, D).astype(jnp.bfloat16)

        s = lax.dot_general(q, k, (((2,), (2,)), ((1,), (1,))),
                            preferred_element_type=jnp.float32)
        s = s * (D ** -0.5)

        base = my_x * NPAGES_LOCAL + i * CP
        slot = lax.broadcasted_iota(jnp.int32, (B, NSLOTS, CP), 1)
        page = lax.broadcasted_iota(jnp.int32, (B, NSLOTS, CP), 2)
        btl = bt_ref[...] - base
        hit = (btl[:, :, None] == page) & (slot < lens_ref[...][:, :, None])
        counts = jnp.sum(hit.astype(jnp.float32), axis=1)

        rk = lax.broadcasted_iota(jnp.int32, (CP, CK), 1) // BS
        rp = lax.broadcasted_iota(jnp.int32, (CP, CK), 0)
        repmat = (rk == rp).astype(jnp.bfloat16)
        ck = lax.dot_general(counts.astype(jnp.bfloat16), repmat,
                             (((1,), (0,)), ((), ())),
                             preferred_element_type=jnp.float32)

        m_prev = m_acc[...]
        m_new = jnp.maximum(m_prev, jnp.max(s, axis=2))
        alpha = jnp.exp(m_prev - m_new)
        p = jnp.exp(s - m_new[:, :, None]) * ck[None, :, :]
        l_acc[...] = l_acc[...] * alpha + jnp.sum(p, axis=2)
        o_acc[...] = o_acc[...] * alpha[:, :, None] + lax.dot_general(
            p.astype(jnp.bfloat16), v, (((2,), (0,)), ((0,), (1,))),
            preferred_element_type=jnp.float32)
        m_acc[...] = m_new

        @pl.when(i == NCHUNK - 1)
        def _():
            ml_send[0, :, :] = m_acc[...]
            ml_send[1, :, :] = l_acc[...]
            pl.semaphore_wait(barrier, 1)
            rdma_ml = pltpu.make_async_remote_copy(
                src_ref=ml_send, dst_ref=ml_recv,
                send_sem=send_sem_ml, recv_sem=recv_sem_ml,
                device_id=nbr, device_id_type=pl.DeviceIdType.MESH)
            rdma_o = pltpu.make_async_remote_copy(
                src_ref=o_acc, dst_ref=o_recv,
                send_sem=send_sem_o, recv_sem=recv_sem_o,
                device_id=nbr, device_id_type=pl.DeviceIdType.MESH)
            rdma_ml.start()
            rdma_o.start()
            rdma_ml.wait()
            rdma_o.wait()

            m_loc = m_acc[...]
            l_loc = l_acc[...]
            m_rem = ml_recv[0, :, :]
            l_rem = ml_recv[1, :, :]
            m = jnp.maximum(m_loc, m_rem)
            a_loc = jnp.exp(m_loc - m)
            a_rem = jnp.exp(m_rem - m)
            l = l_loc * a_loc + l_rem * a_rem
            o = o_acc[...] * a_loc[:, :, None] + o_recv[...] * a_rem[:, :, None]
            o = o / l[:, :, None]
            out_ref[...] = jnp.swapaxes(o, 0, 1).reshape(B, 1, H, D)

    return pl.pallas_call(
        body,
        grid=(NCHUNK,),
        out_shape=jax.ShapeDtypeStruct((B, 1, H, D), jnp.float32),
        in_specs=[
            pl.BlockSpec((B, 1, H, D), lambda i: (0, 0, 0, 0)),
            pl.BlockSpec((CP, BS, H, D), lambda i: (i, 0, 0, 0)),
            pl.BlockSpec((CP, BS, H, D), lambda i: (i, 0, 0, 0)),
            pl.BlockSpec((B, NSLOTS), lambda i: (0, 0)),
            pl.BlockSpec((B, 1), lambda i: (0, 0)),
        ],
        out_specs=pl.BlockSpec((B, 1, H, D), lambda i: (0, 0, 0, 0)),
        scratch_shapes=[
            pltpu.VMEM((H, B), jnp.float32),
            pltpu.VMEM((H, B), jnp.float32),
            pltpu.VMEM((H, B, D), jnp.float32),
            pltpu.VMEM((2, H, B), jnp.float32),
            pltpu.VMEM((2, H, B), jnp.float32),
            pltpu.VMEM((H, B, D), jnp.float32),
            pltpu.SemaphoreType.DMA,
            pltpu.SemaphoreType.DMA,
            pltpu.SemaphoreType.DMA,
            pltpu.SemaphoreType.DMA,
        ],
        compiler_params=pltpu.CompilerParams(collective_id=0),
    )(Q, K, V, bt, lens2)


# device time: 63851 ns/iter; 1.0221x vs baseline; 1.0221x over previous
import jax
import jax.numpy as jnp
from jax import lax
from jax.experimental import pallas as pl
from jax.experimental.pallas import tpu as pltpu

B, H, D, BS = 16, 16, 64, 16
NSLOTS = 128
NPAGES_LOCAL = 128
NKEYS = NPAGES_LOCAL * BS
HP = 2
NSTEP = H // HP
W = HP * D


def kernel(Q, K, V, bt, lens):
    lens2 = lens.reshape(B, 1)
    q2 = Q.reshape(B, H * D)
    k2 = K.reshape(NKEYS, H * D)
    v2 = V.reshape(NKEYS, H * D)

    def body(q_ref, k_ref, v_ref, bt_ref, lens_ref, out_ref,
             ck_s, msend, lsend, osend, mrecv, lrecv, orecv,
             sem_sm, sem_rm, sem_sl, sem_rl, sem_so, sem_ro):
        i = pl.program_id(0)
        my_x = lax.axis_index("x")
        my_y = lax.axis_index("y")
        nbr = (1 - my_x, my_y)
        barrier = pltpu.get_barrier_semaphore()

        @pl.when(i == 0)
        def _():
            pl.semaphore_signal(barrier, inc=1, device_id=nbr,
                                device_id_type=pl.DeviceIdType.MESH)
            slot = lax.broadcasted_iota(
                jnp.int32, (B, NSLOTS, NPAGES_LOCAL), 1)
            page = lax.broadcasted_iota(
                jnp.int32, (B, NSLOTS, NPAGES_LOCAL), 2)
            btl = bt_ref[...] - my_x * NPAGES_LOCAL
            hit = ((btl[:, :, None] == page)
                   & (slot < lens_ref[...][:, :, None]))
            counts = jnp.sum(hit.astype(jnp.float32), axis=1)
            rk = lax.broadcasted_iota(jnp.int32, (NPAGES_LOCAL, NKEYS), 1)
            rp = lax.broadcasted_iota(jnp.int32, (NPAGES_LOCAL, NKEYS), 0)
            repmat = (rk // BS == rp).astype(jnp.bfloat16)
            ck = lax.dot_general(counts.astype(jnp.bfloat16), repmat,
                                 (((1,), (0,)), ((), ())),
                                 preferred_element_type=jnp.float32)
            ck_s[0:B, :] = ck
            ck_s[B:2 * B, :] = ck

        qw = q_ref[...]
        lane = lax.broadcasted_iota(jnp.int32, (B, W), 1)
        q_lo = jnp.where(lane < D, qw, 0.0)
        q_hi = jnp.where(lane >= D, qw, 0.0)
        qd = jnp.concatenate([q_lo, q_hi], 0).astype(jnp.bfloat16)

        kw = k_ref[...].astype(jnp.bfloat16)
        s = lax.dot_general(qd, kw, (((1,), (1,)), ((), ())),
                            preferred_element_type=jnp.float32)
        s = s * (D ** -0.5)

        m = jnp.max(s, axis=1, keepdims=True)
        p = jnp.exp(s - m) * ck_s[...]
        l = jnp.sum(p, axis=1, keepdims=True)
        vw = v_ref[...].astype(jnp.bfloat16)
        o = lax.dot_general(p.astype(jnp.bfloat16), vw,
                            (((1,), (0,)), ((), ())),
                            preferred_element_type=jnp.float32)

        for hh in range(HP):
            idx = HP * i + hh
            rows, cols = hh * B, hh * D
            msend[pl.ds(idx, 1), :] = jnp.swapaxes(
                m[rows:rows + B, :], 0, 1)
            lsend[pl.ds(idx, 1), :] = jnp.swapaxes(
                l[rows:rows + B, :], 0, 1)
            osend[pl.ds(idx, 1), :, :] = (
                o[rows:rows + B, cols:cols + D].reshape(1, B, D))

        @pl.when(i == NSTEP - 1)
        def _():
            pl.semaphore_wait(barrier, 1)
            rdma_m = pltpu.make_async_remote_copy(
                src_ref=msend, dst_ref=mrecv, send_sem=sem_sm,
                recv_sem=sem_rm, device_id=nbr,
                device_id_type=pl.DeviceIdType.MESH)
            rdma_l = pltpu.make_async_remote_copy(
                src_ref=lsend, dst_ref=lrecv, send_sem=sem_sl,
                recv_sem=sem_rl, device_id=nbr,
                device_id_type=pl.DeviceIdType.MESH)
            rdma_o = pltpu.make_async_remote_copy(
                src_ref=osend, dst_ref=orecv, send_sem=sem_so,
                recv_sem=sem_ro, device_id=nbr,
                device_id_type=pl.DeviceIdType.MESH)
            rdma_m.start()
            rdma_l.start()
            rdma_o.start()
            rdma_m.wait()
            rdma_l.wait()
            rdma_o.wait()

            m_loc, l_loc = msend[...], lsend[...]
            m_rem, l_rem = mrecv[...], lrecv[...]
            mm = jnp.maximum(m_loc, m_rem)
            a_loc = jnp.exp(m_loc - mm)
            a_rem = jnp.exp(m_rem - mm)
            ll = l_loc * a_loc + l_rem * a_rem
            oo = (osend[...] * a_loc[:, :, None]
                  + orecv[...] * a_rem[:, :, None]) / ll[:, :, None]
            out_ref[...] = jnp.swapaxes(oo, 0, 1).reshape(B, 1, H, D)

    return pl.pallas_call(
        body,
        grid=(NSTEP,),
        out_shape=jax.ShapeDtypeStruct((B, 1, H, D), jnp.float32),
        in_specs=[
            pl.BlockSpec((B, W), lambda i: (0, i)),
            pl.BlockSpec((NKEYS, W), lambda i: (0, i)),
            pl.BlockSpec((NKEYS, W), lambda i: (0, i)),
            pl.BlockSpec((B, NSLOTS), lambda i: (0, 0)),
            pl.BlockSpec((B, 1), lambda i: (0, 0)),
        ],
        out_specs=pl.BlockSpec((B, 1, H, D), lambda i: (0, 0, 0, 0)),
        scratch_shapes=[
            pltpu.VMEM((2 * B, NKEYS), jnp.float32),
            pltpu.VMEM((H, B), jnp.float32),
            pltpu.VMEM((H, B), jnp.float32),
            pltpu.VMEM((H, B, D), jnp.float32),
            pltpu.VMEM((H, B), jnp.float32),
            pltpu.VMEM((H, B), jnp.float32),
            pltpu.VMEM((H, B, D), jnp.float32),
            pltpu.SemaphoreType.DMA,
            pltpu.SemaphoreType.DMA,
            pltpu.SemaphoreType.DMA,
            pltpu.SemaphoreType.DMA,
            pltpu.SemaphoreType.DMA,
            pltpu.SemaphoreType.DMA,
        ],
        compiler_params=pltpu.CompilerParams(collective_id=0),
    )(q2, k2, v2, bt, lens2)


# device time: 21744 ns/iter; 3.0012x vs baseline; 2.9365x over previous
import jax
import jax.numpy as jnp
from jax import lax
from jax.experimental import pallas as pl
from jax.experimental.pallas import tpu as pltpu

B, H, D, BS = 16, 16, 64, 16
NSLOTS = 128
NP = 128
R = H * B
HD = H * D


def kernel(Q, K, V, bt, lens):
    lens2 = lens.reshape(B, 1)
    q2 = Q.reshape(B, HD)
    k3 = K.transpose(1, 2, 3, 0).reshape(BS, HD, NP)
    v3 = V.transpose(1, 2, 3, 0).reshape(BS, HD, NP)

    def body(q_ref, k_ref, v_ref, bt_ref, lens_ref, out_ref,
             s_ref, msend, lsend, osend, mrecv, lrecv, orecv,
             sem_sm, sem_rm, sem_sl, sem_rl, sem_so, sem_ro):
        my_x = lax.axis_index("x")
        my_y = lax.axis_index("y")
        nbr = (1 - my_x, my_y)
        barrier = pltpu.get_barrier_semaphore()
        pl.semaphore_signal(barrier, inc=1, device_id=nbr,
                            device_id_type=pl.DeviceIdType.MESH)

        slot = lax.broadcasted_iota(jnp.int32, (B, NSLOTS, NP), 1)
        page = lax.broadcasted_iota(jnp.int32, (B, NSLOTS, NP), 2)
        btl = bt_ref[...] - my_x * NP
        hit = (btl[:, :, None] == page) & (slot < lens_ref[...][:, :, None])
        counts = jnp.sum(hit.astype(jnp.float32), axis=1)

        qrep = jnp.concatenate([q_ref[...]] * H, axis=0)
        rowh = lax.broadcasted_iota(jnp.int32, (R, HD), 0) // B
        colh = lax.broadcasted_iota(jnp.int32, (R, HD), 1) // D
        qbig = jnp.where(rowh == colh, qrep, 0.0).astype(jnp.bfloat16)

        kb = k_ref[...].astype(jnp.bfloat16)
        vb = v_ref[...].astype(jnp.bfloat16)

        for t in range(BS):
            s_ref[t, :, :] = lax.dot_general(
                qbig, kb[t], (((1,), (0,)), ((), ())),
                preferred_element_type=jnp.float32) * (D ** -0.5)

        s4 = s_ref[...].reshape(BS, H, B, NP)
        m_hb = jnp.max(jnp.max(s4, axis=3), axis=0)
        p4 = jnp.exp(s4 - m_hb[None, :, :, None]) * counts[None, None, :, :]
        l_hb = jnp.sum(jnp.sum(p4, axis=3), axis=0)
        msend[...] = m_hb
        lsend[...] = l_hb

        pb = p4.reshape(BS, R, NP).astype(jnp.bfloat16)
        o_big = lax.dot_general(pb[0], vb[0], (((1,), (1,)), ((), ())),
                                preferred_element_type=jnp.float32)
        for t in range(1, BS):
            o_big = o_big + lax.dot_general(
                pb[t], vb[t], (((1,), (1,)), ((), ())),
                preferred_element_type=jnp.float32)

        for h in range(H):
            osend[h, :, :] = o_big[h * B:(h + 1) * B, h * D:(h + 1) * D]

        pl.semaphore_wait(barrier, 1)
        rdma_m = pltpu.make_async_remote_copy(
            src_ref=msend, dst_ref=mrecv, send_sem=sem_sm,
            recv_sem=sem_rm, device_id=nbr,
            device_id_type=pl.DeviceIdType.MESH)
        rdma_l = pltpu.make_async_remote_copy(
            src_ref=lsend, dst_ref=lrecv, send_sem=sem_sl,
            recv_sem=sem_rl, device_id=nbr,
            device_id_type=pl.DeviceIdType.MESH)
        rdma_o = pltpu.make_async_remote_copy(
            src_ref=osend, dst_ref=orecv, send_sem=sem_so,
            recv_sem=sem_ro, device_id=nbr,
            device_id_type=pl.DeviceIdType.MESH)
        rdma_m.start()
        rdma_l.start()
        rdma_o.start()
        rdma_m.wait()
        rdma_l.wait()
        rdma_o.wait()

        m_rem, l_rem = mrecv[...], lrecv[...]
        mm = jnp.maximum(m_hb, m_rem)
        a_loc = jnp.exp(m_hb - mm)
        a_rem = jnp.exp(m_rem - mm)
        ll = l_hb * a_loc + l_rem * a_rem
        oo = (osend[...] * a_loc[:, :, None]
              + orecv[...] * a_rem[:, :, None]) / ll[:, :, None]
        out_ref[...] = jnp.swapaxes(oo, 0, 1).reshape(B, 1, H, D)

    return pl.pallas_call(
        body,
        out_shape=jax.ShapeDtypeStruct((B, 1, H, D), jnp.float32),
        in_specs=[pl.BlockSpec(memory_space=pltpu.VMEM)] * 5,
        out_specs=pl.BlockSpec(memory_space=pltpu.VMEM),
        scratch_shapes=[
            pltpu.VMEM((BS, R, NP), jnp.float32),
            pltpu.VMEM((H, B), jnp.float32),
            pltpu.VMEM((H, B), jnp.float32),
            pltpu.VMEM((H, B, D), jnp.float32),
            pltpu.VMEM((H, B), jnp.float32),
            pltpu.VMEM((H, B), jnp.float32),
            pltpu.VMEM((H, B, D), jnp.float32),
            pltpu.SemaphoreType.DMA,
            pltpu.SemaphoreType.DMA,
            pltpu.SemaphoreType.DMA,
            pltpu.SemaphoreType.DMA,
            pltpu.SemaphoreType.DMA,
            pltpu.SemaphoreType.DMA,
        ],
        compiler_params=pltpu.CompilerParams(
            collective_id=0, vmem_limit_bytes=100 * 1024 * 1024),
    )(q2, k3, v3, bt, lens2)


# device time: 19400 ns/iter; 3.3639x vs baseline; 1.1208x over previous
import jax
import jax.numpy as jnp
from jax import lax
from jax.experimental import pallas as pl
from jax.experimental.pallas import tpu as pltpu

B, H, D, BS = 16, 16, 64, 16
NSLOTS = 128
NP = 128
R = H * B
HD = H * D
G = 4
HG = H // G
CW = HG * D
RG = HG * B


def kernel(Q, K, V, bt, lens):
    lens2 = lens.reshape(B, 1)
    q2 = Q.reshape(B, HD)
    k3 = K.transpose(1, 2, 3, 0).reshape(BS, HD, NP)
    v3 = V.transpose(1, 2, 3, 0).reshape(BS, HD, NP)

    def body(q_ref, k_ref, v_ref, bt_ref, lens_ref, out_ref,
             s_ref, msend, lsend, osend, mrecv, lrecv, orecv,
             sem_sm, sem_rm, sem_sl, sem_rl, sem_so, sem_ro):
        my_x = lax.axis_index("x")
        my_y = lax.axis_index("y")
        nbr = (1 - my_x, my_y)
        barrier = pltpu.get_barrier_semaphore()
        pl.semaphore_signal(barrier, inc=1, device_id=nbr,
                            device_id_type=pl.DeviceIdType.MESH)

        slot = lax.broadcasted_iota(jnp.int32, (B, NSLOTS, NP), 1)
        page = lax.broadcasted_iota(jnp.int32, (B, NSLOTS, NP), 2)
        btl = bt_ref[...] - my_x * NP
        hit = (btl[:, :, None] == page) & (slot < lens_ref[...][:, :, None])
        counts = jnp.sum(hit.astype(jnp.float32), axis=1)

        qbigs = []
        for g in range(G):
            q_g = q_ref[:, g * CW:(g + 1) * CW]
            qrep = jnp.concatenate([q_g] * HG, axis=0)
            rowh = lax.broadcasted_iota(jnp.int32, (RG, CW), 0) // B
            colh = lax.broadcasted_iota(jnp.int32, (RG, CW), 1) // D
            qbigs.append(jnp.where(rowh == colh, qrep, 0.0)
                         .astype(jnp.bfloat16))

        kb = k_ref[...].astype(jnp.bfloat16)
        vb = v_ref[...].astype(jnp.bfloat16)

        for t in range(BS):
            for g in range(G):
                s_ref[t, g * RG:(g + 1) * RG, :] = lax.dot_general(
                    qbigs[g], kb[t, g * CW:(g + 1) * CW, :],
                    (((1,), (0,)), ((), ())),
                    preferred_element_type=jnp.float32) * (D ** -0.5)

        s4 = s_ref[...].reshape(BS, H, B, NP)
        m_hb = jnp.max(jnp.max(s4, axis=3), axis=0)
        p4 = jnp.exp(s4 - m_hb[None, :, :, None]) * counts[None, None, :, :]
        l_hb = jnp.sum(jnp.sum(p4, axis=3), axis=0)
        msend[...] = m_hb
        lsend[...] = l_hb

        pb = p4.reshape(BS, R, NP).astype(jnp.bfloat16)
        for g in range(G):
            o_g = lax.dot_general(
                pb[0, g * RG:(g + 1) * RG, :],
                vb[0, g * CW:(g + 1) * CW, :],
                (((1,), (1,)), ((), ())),
                preferred_element_type=jnp.float32)
            for t in range(1, BS):
                o_g = o_g + lax.dot_general(
                    pb[t, g * RG:(g + 1) * RG, :],
                    vb[t, g * CW:(g + 1) * CW, :],
                    (((1,), (1,)), ((), ())),
                    preferred_element_type=jnp.float32)
            for hl in range(HG):
                osend[g * HG + hl, :, :] = (
                    o_g[hl * B:(hl + 1) * B, hl * D:(hl + 1) * D])

        pl.semaphore_wait(barrier, 1)
        rdma_m = pltpu.make_async_remote_copy(
            src_ref=msend, dst_ref=mrecv, send_sem=sem_sm,
            recv_sem=sem_rm, device_id=nbr,
            device_id_type=pl.DeviceIdType.MESH)
        rdma_l = pltpu.make_async_remote_copy(
            src_ref=lsend, dst_ref=lrecv, send_sem=sem_sl,
            recv_sem=sem_rl, device_id=nbr,
            device_id_type=pl.DeviceIdType.MESH)
        rdma_o = pltpu.make_async_remote_copy(
            src_ref=osend, dst_ref=orecv, send_sem=sem_so,
            recv_sem=sem_ro, device_id=nbr,
            device_id_type=pl.DeviceIdType.MESH)
        rdma_m.start()
        rdma_l.start()
        rdma_o.start()
        rdma_m.wait()
        rdma_l.wait()
        rdma_o.wait()

        m_rem, l_rem = mrecv[...], lrecv[...]
        mm = jnp.maximum(m_hb, m_rem)
        a_loc = jnp.exp(m_hb - mm)
        a_rem = jnp.exp(m_rem - mm)
        ll = l_hb * a_loc + l_rem * a_rem
        oo = (osend[...] * a_loc[:, :, None]
              + orecv[...] * a_rem[:, :, None]) / ll[:, :, None]
        out_ref[...] = jnp.swapaxes(oo, 0, 1).reshape(B, 1, H, D)

    return pl.pallas_call(
        body,
        out_shape=jax.ShapeDtypeStruct((B, 1, H, D), jnp.float32),
        in_specs=[pl.BlockSpec(memory_space=pltpu.VMEM)] * 5,
        out_specs=pl.BlockSpec(memory_space=pltpu.VMEM),
        scratch_shapes=[
            pltpu.VMEM((BS, R, NP), jnp.float32),
            pltpu.VMEM((H, B), jnp.float32),
            pltpu.VMEM((H, B), jnp.float32),
            pltpu.VMEM((H, B, D), jnp.float32),
            pltpu.VMEM((H, B), jnp.float32),
            pltpu.VMEM((H, B), jnp.float32),
            pltpu.VMEM((H, B, D), jnp.float32),
            pltpu.SemaphoreType.DMA,
            pltpu.SemaphoreType.DMA,
            pltpu.SemaphoreType.DMA,
            pltpu.SemaphoreType.DMA,
            pltpu.SemaphoreType.DMA,
            pltpu.SemaphoreType.DMA,
        ],
        compiler_params=pltpu.CompilerParams(
            collective_id=0, vmem_limit_bytes=100 * 1024 * 1024),
    )(q2, k3, v3, bt, lens2)
